# baseline (speedup 1.0000x reference)
import jax
import jax.numpy as jnp
from jax import lax
from jax.experimental import pallas as pl
from jax.experimental.pallas import tpu as pltpu
from jax.experimental.pallas import tpu_sc as plsc

_N = 100
_NP = 128
_E = 6400
_ER = 50
_EC = 128
_ACC = _NP * _NP
_SLICE = _ACC // 16
_F32 = jnp.float32
_PH = lax.Precision.HIGHEST


def _sc_body(ei_hbm, ew_hbm, out_hbm, src_v, dst_v, w_v, idx_v, z_v, acc_sh):
    c = lax.axis_index("c")
    s = lax.axis_index("s")
    wid = c * 16 + s
    for i in range(_SLICE // 16):
        z_v[pl.ds(i * 16, 16)] = jnp.zeros((16,), _F32)
    pltpu.sync_copy(z_v, acc_sh.at[pl.ds(s * _SLICE, _SLICE)])
    plsc.subcore_barrier()

    def do_row(j, r):
        pltpu.sync_copy(ei_hbm.at[0, r], src_v.at[j])
        pltpu.sync_copy(ei_hbm.at[1, r], dst_v.at[j])
        pltpu.sync_copy(ew_hbm.at[r], w_v.at[j])
        for k in range(_EC // 16):
            sl = pl.ds(k * 16, 16)
            idx_v[j, sl] = dst_v[j, sl] * _NP + src_v[j, sl]
        pltpu.sync_copy(w_v.at[j], acc_sh.at[idx_v.at[j]], add=True)

    do_row(0, wid)

    @pl.when(wid < _ER - 32)
    def _():
        do_row(1, wid + 32)

    plsc.subcore_barrier()
    pltpu.sync_copy(acc_sh.at[pl.ds(s * _SLICE, _SLICE)],
                    out_hbm.at[c, pl.ds(s * _SLICE, _SLICE)])


_sc_build = pl.kernel(
    _sc_body,
    mesh=plsc.VectorSubcoreMesh(core_axis_name="c", subcore_axis_name="s"),
    out_type=jax.ShapeDtypeStruct((2, _ACC), _F32),
    scratch_types=[
        pltpu.VMEM((2, _EC), jnp.int32),
        pltpu.VMEM((2, _EC), jnp.int32),
        pltpu.VMEM((2, _EC), _F32),
        pltpu.VMEM((2, _EC), jnp.int32),
        pltpu.VMEM((_SLICE,), _F32),
        pltpu.VMEM_SHARED((_ACC,), _F32),
    ],
)


def _dense_body(a2_ref, xv_ref, wg1_ref, bg1_ref,
                wg2_ref, bg2_ref, wg3_ref, bg3_ref, gam_ref, bet_ref,
                w1_ref, b1_ref, w2r_ref, b2_ref, out_ref):
    at = a2_ref[0] + a2_ref[1]
    deg = jnp.sum(at, axis=1, keepdims=True) + 1.0
    dinv = lax.rsqrt(deg)
    xp = jnp.concatenate(
        [xv_ref[...], jnp.zeros((_NP - _N, 128), _F32)], axis=0)

    def gcn(h, w, b):
        z = jnp.dot(h, w, preferred_element_type=_F32)
        zh = dinv * z
        agg = jnp.dot(at, zh, precision=_PH, preferred_element_type=_F32) + zh
        return jax.nn.relu(dinv * agg + b)

    h = gcn(xp, wg1_ref[...], bg1_ref[...])
    h = gcn(h, wg2_ref[...], bg2_ref[...])
    h = gcn(h, wg3_ref[...], bg3_ref[...])

    rmask = (lax.broadcasted_iota(jnp.int32, (_NP, 1), 0) < _N).astype(_F32)
    mean = jnp.sum(h * rmask, axis=0, keepdims=True) * (1.0 / _N)
    diff = h - mean
    var = jnp.sum(diff * diff * rmask, axis=0, keepdims=True) * (1.0 / _N)
    hn = diff * lax.rsqrt(var + 1e-5) * gam_ref[...] + bet_ref[...]

    l = jax.nn.relu(jnp.dot(hn, w1_ref[...],
                            preferred_element_type=_F32) + b1_ref[...])
    g = lax.dot_general(l[:_N, :], w2r_ref[...], (((0,), (0,)), ((), ())),
                        preferred_element_type=_F32)
    acc = b2_ref[...]
    for c in range(10):
        acc = acc + g[c:c + 1, c * 128:(c + 1) * 128]
    out_ref[...] = acc


def kernel(x, edge_index, edge_attr, Wg1, bg1, Wg2, bg2, Wg3, bg3,
           gamma, beta, W1, b1, W2, b2):
    ei = edge_index.astype(jnp.int32).reshape(2, _ER, _EC)
    ew = edge_attr.reshape(_ER, _EC)
    a2 = _sc_build(ei, ew).reshape(2, _NP, _NP)
    out = pl.pallas_call(
        _dense_body,
        out_shape=jax.ShapeDtypeStruct((1, 128), _F32),
    )(a2, x.reshape(_N, 128),
      Wg1, bg1.reshape(1, 64), Wg2, bg2.reshape(1, 128),
      Wg3, bg3.reshape(1, 256), gamma.reshape(1, 256), beta.reshape(1, 256),
      W1, b1.reshape(1, 10), W2.reshape(_N, 1280), b2.reshape(1, 128))
    return out.reshape(128)


# speedup vs baseline: 1.0825x; 1.0825x over previous
import jax
import jax.numpy as jnp
from jax import lax
from jax.experimental import pallas as pl
from jax.experimental.pallas import tpu as pltpu
from jax.experimental.pallas import tpu_sc as plsc

_N = 100
_NP = 128
_E = 6400
_ER = 50
_EC = 128
_ACC = _NP * _NP
_SLICE = _ACC // 16
_F32 = jnp.float32
_PH = lax.Precision.HIGHEST


def _sc_body(ei_hbm, ew_hbm, out_hbm, src_v, dst_v, w_v, idx_v, z_v,
             acc_sh, sem):
    c = lax.axis_index("c")
    s = lax.axis_index("s")
    wid = c * 16 + s
    for i in range(_SLICE // 16):
        z_v[pl.ds(i * 16, 16)] = jnp.zeros((16,), _F32)
    pltpu.sync_copy(z_v, acc_sh.at[pl.ds(s * _SLICE, _SLICE)])
    plsc.subcore_barrier()

    def fetch_row(j, r):
        return [
            pltpu.async_copy(ei_hbm.at[0, 0, pl.ds(r * _EC, _EC)],
                             src_v.at[j], sem),
            pltpu.async_copy(ei_hbm.at[0, 1, pl.ds(r * _EC, _EC)],
                             dst_v.at[j], sem),
            pltpu.async_copy(ew_hbm.at[0, pl.ds(r * _EC, _EC)],
                             w_v.at[j], sem),
        ]

    def scatter_row(j):
        for k in range(_EC // 16):
            sl = pl.ds(k * 16, 16)
            idx_v[j, sl] = dst_v[j, sl] * _NP + src_v[j, sl]
        pltpu.sync_copy(w_v.at[j], acc_sh.at[idx_v.at[j]], add=True)

    two_rows = wid < _ER - 32
    cps0 = fetch_row(0, wid)

    @pl.when(two_rows)
    def _():
        for cp in fetch_row(1, wid + 32):
            cp.wait()

    for cp in cps0:
        cp.wait()
    scatter_row(0)

    @pl.when(two_rows)
    def _():
        scatter_row(1)

    plsc.subcore_barrier()
    ocps = [
        pltpu.async_copy(acc_sh.at[pl.ds(s * _SLICE + k * _NP, _NP)],
                         out_hbm.at[c, s * 8 + k, pl.ds(0, _NP)], sem)
        for k in range(8)
    ]
    for cp in ocps:
        cp.wait()


_sc_build = pl.kernel(
    _sc_body,
    mesh=plsc.VectorSubcoreMesh(core_axis_name="c", subcore_axis_name="s"),
    out_type=jax.ShapeDtypeStruct((2, _NP, _NP), _F32),
    scratch_types=[
        pltpu.VMEM((2, _EC), jnp.int32),
        pltpu.VMEM((2, _EC), jnp.int32),
        pltpu.VMEM((2, _EC), _F32),
        pltpu.VMEM((2, _EC), jnp.int32),
        pltpu.VMEM((_SLICE,), _F32),
        pltpu.VMEM_SHARED((_ACC,), _F32),
        pltpu.SemaphoreType.DMA,
    ],
)


def _dense_body(a2_ref, x_ref, wg1_ref, bg1_ref,
                wg2_ref, bg2_ref, wg3_ref, bg3_ref, gam_ref, bet_ref,
                w1_ref, b1_ref, w2_ref, b2_ref, out_ref):
    at = a2_ref[0] + a2_ref[1]
    deg = jnp.sum(at, axis=1, keepdims=True) + 1.0
    dinv = lax.rsqrt(deg)
    xp = jnp.concatenate(
        [x_ref[0], jnp.zeros((_NP - _N, 128), _F32)], axis=0)

    def gcn(h, w, b):
        z = jnp.dot(h, w, preferred_element_type=_F32)
        zh = dinv * z
        agg = jnp.dot(at, zh, precision=_PH, preferred_element_type=_F32) + zh
        return jax.nn.relu(dinv * agg + b)

    h = gcn(xp, wg1_ref[...], bg1_ref[...])
    h = gcn(h, wg2_ref[...], bg2_ref[...])
    h = gcn(h, wg3_ref[...], bg3_ref[...])

    rmask = (lax.broadcasted_iota(jnp.int32, (_NP, 1), 0) < _N).astype(_F32)
    mean = jnp.sum(h * rmask, axis=0, keepdims=True) * (1.0 / _N)
    diff = h - mean
    var = jnp.sum(diff * diff * rmask, axis=0, keepdims=True) * (1.0 / _N)
    hn = diff * lax.rsqrt(var + 1e-5) * gam_ref[...] + bet_ref[...]

    l = jax.nn.relu(jnp.dot(hn, w1_ref[...],
                            preferred_element_type=_F32) + b1_ref[...])
    r_i = lax.broadcasted_iota(jnp.int32, (1000, _N), 0)
    i_i = lax.broadcasted_iota(jnp.int32, (1000, _N), 1)
    nsel = (r_i // 10 == i_i).astype(_F32)
    r_c = lax.broadcasted_iota(jnp.int32, (1000, 10), 0)
    c_c = lax.broadcasted_iota(jnp.int32, (1000, 10), 1)
    qsel = (r_c % 10 == c_c).astype(_F32)
    lrows = jnp.dot(nsel, l[:_N, :], precision=_PH,
                    preferred_element_type=_F32)
    s = jnp.sum(lrows * qsel, axis=1, keepdims=True)
    out = lax.dot_general(s, w2_ref[...], (((0,), (0,)), ((), ())),
                          preferred_element_type=_F32)
    out_ref[...] = out[0] + b2_ref[...]


def kernel(x, edge_index, edge_attr, Wg1, bg1, Wg2, bg2, Wg3, bg3,
           gamma, beta, W1, b1, W2, b2):
    a2 = _sc_build(edge_index.astype(jnp.int32), edge_attr)
    return pl.pallas_call(
        _dense_body,
        out_shape=jax.ShapeDtypeStruct((128,), _F32),
    )(a2, x, Wg1, bg1, Wg2, bg2, Wg3, bg3, gamma, beta, W1, b1, W2, b2)


# speedup vs baseline: 1.1885x; 1.0979x over previous
import jax
import jax.numpy as jnp
from jax import lax
from jax.experimental import pallas as pl
from jax.experimental.pallas import tpu as pltpu
from jax.experimental.pallas import tpu_sc as plsc

_N = 100
_NP = 128
_E = 6400
_ER = 50
_EC = 128
_ACC = _NP * _NP
_SLICE = _ACC // 16
_F32 = jnp.float32
_PH = lax.Precision.HIGHEST


def _sc_body(ei_hbm, ew_hbm, out_hbm, src_v, dst_v, w_v, idx_v, z_v,
             acc_sh, sem):
    s = lax.axis_index("s")
    for i in range(_SLICE // 16):
        z_v[pl.ds(i * 16, 16)] = jnp.zeros((16,), _F32)
    pltpu.sync_copy(z_v, acc_sh.at[pl.ds(s * _SLICE, _SLICE)])
    plsc.subcore_barrier()

    def fetch_pair(half, r0):
        o = half * 2 * _EC
        sl = pl.ds(r0 * _EC, 2 * _EC)
        return [
            pltpu.async_copy(ei_hbm.at[0, 0, sl], src_v.at[pl.ds(o, 2 * _EC)], sem),
            pltpu.async_copy(ei_hbm.at[0, 1, sl], dst_v.at[pl.ds(o, 2 * _EC)], sem),
            pltpu.async_copy(ew_hbm.at[0, sl], w_v.at[pl.ds(o, 2 * _EC)], sem),
        ]

    def scatter_rows(j0):
        for j in (j0, j0 + 1):
            for k in range(_EC // 16):
                sl = pl.ds(j * _EC + k * 16, 16)
                idx_v[j, pl.ds(k * 16, 16)] = dst_v[sl] * _NP + src_v[sl]
            pltpu.sync_copy(w_v.at[pl.ds(j * _EC, _EC)],
                            acc_sh.at[idx_v.at[j]], add=True)

    second = s < _ER // 2 - 16
    cps0 = fetch_pair(0, 2 * s)

    @pl.when(second)
    def _():
        fetch_pair(1, 32 + 2 * s)

    for cp in cps0:
        cp.wait()
    scatter_rows(0)

    @pl.when(second)
    def _():
        o = 2 * _EC
        sl = pl.ds((32 + 2 * s) * _EC, 2 * _EC)
        pltpu.make_async_copy(ei_hbm.at[0, 0, sl],
                              src_v.at[pl.ds(o, 2 * _EC)], sem).wait()
        pltpu.make_async_copy(ei_hbm.at[0, 1, sl],
                              dst_v.at[pl.ds(o, 2 * _EC)], sem).wait()
        pltpu.make_async_copy(ew_hbm.at[0, sl],
                              w_v.at[pl.ds(o, 2 * _EC)], sem).wait()
        scatter_rows(2)

    plsc.subcore_barrier()
    ocps = [
        pltpu.async_copy(acc_sh.at[pl.ds(s * _SLICE + k * _NP, _NP)],
                         out_hbm.at[s * 8 + k, pl.ds(0, _NP)], sem)
        for k in range(8)
    ]
    for cp in ocps:
        cp.wait()


_sc_build = pl.kernel(
    _sc_body,
    mesh=plsc.VectorSubcoreMesh(core_axis_name="c", subcore_axis_name="s",
                                num_cores=1),
    out_type=jax.ShapeDtypeStruct((_NP, _NP), _F32),
    scratch_types=[
        pltpu.VMEM((4 * _EC,), jnp.int32),
        pltpu.VMEM((4 * _EC,), jnp.int32),
        pltpu.VMEM((4 * _EC,), _F32),
        pltpu.VMEM((4, _EC), jnp.int32),
        pltpu.VMEM((_SLICE,), _F32),
        pltpu.VMEM_SHARED((_ACC,), _F32),
        pltpu.SemaphoreType.DMA,
    ],
)


def _dense_body(a_ref, x_ref, wg1_ref, bg1_ref,
                wg2_ref, bg2_ref, wg3_ref, bg3_ref, gam_ref, bet_ref,
                w1_ref, b1_ref, w2_ref, b2_ref, out_ref):
    at = a_ref[...]
    deg = jnp.sum(at, axis=1, keepdims=True) + 1.0
    dinv = lax.rsqrt(deg)
    xp = jnp.concatenate(
        [x_ref[0], jnp.zeros((_NP - _N, 128), _F32)], axis=0)

    def gcn(h, w, b):
        z = jnp.dot(h, w, preferred_element_type=_F32)
        zh = dinv * z
        agg = jnp.dot(at, zh, precision=_PH, preferred_element_type=_F32) + zh
        return jax.nn.relu(dinv * agg + b)

    h = gcn(xp, wg1_ref[...], bg1_ref[...])
    h = gcn(h, wg2_ref[...], bg2_ref[...])
    h = gcn(h, wg3_ref[...], bg3_ref[...])

    rmask = (lax.broadcasted_iota(jnp.int32, (_NP, 1), 0) < _N).astype(_F32)
    mean = jnp.sum(h * rmask, axis=0, keepdims=True) * (1.0 / _N)
    diff = h - mean
    var = jnp.sum(diff * diff * rmask, axis=0, keepdims=True) * (1.0 / _N)
    hn = diff * lax.rsqrt(var + 1e-5) * gam_ref[...] + bet_ref[...]

    l = jax.nn.relu(jnp.dot(hn, w1_ref[...],
                            preferred_element_type=_F32) + b1_ref[...])
    r_i = lax.broadcasted_iota(jnp.int32, (1000, _N), 0)
    i_i = lax.broadcasted_iota(jnp.int32, (1000, _N), 1)
    nsel = (r_i // 10 == i_i).astype(_F32)
    r_c = lax.broadcasted_iota(jnp.int32, (1000, 10), 0)
    c_c = lax.broadcasted_iota(jnp.int32, (1000, 10), 1)
    qsel = (r_c % 10 == c_c).astype(_F32)
    lrows = jnp.dot(nsel, l[:_N, :], preferred_element_type=_F32)
    sflat = jnp.sum(lrows * qsel, axis=1, keepdims=True)
    out = jnp.sum(sflat * w2_ref[...], axis=0, keepdims=True)
    out_ref[...] = out[0] + b2_ref[...]


def kernel(x, edge_index, edge_attr, Wg1, bg1, Wg2, bg2, Wg3, bg3,
           gamma, beta, W1, b1, W2, b2):
    a = _sc_build(edge_index.astype(jnp.int32), edge_attr)
    return pl.pallas_call(
        _dense_body,
        out_shape=jax.ShapeDtypeStruct((128,), _F32),
    )(a, x, Wg1, bg1, Wg2, bg2, Wg3, bg3, gamma, beta, W1, b1, W2, b2)


# speedup vs baseline: 1.1902x; 1.0015x over previous
import jax
import jax.numpy as jnp
from jax import lax
from jax.experimental import pallas as pl
from jax.experimental.pallas import tpu as pltpu
from jax.experimental.pallas import tpu_sc as plsc

_N = 100
_NP = 128
_E = 6400
_ER = 50
_EC = 128
_ACC = _NP * _NP
_SLICE = _ACC // 16
_F32 = jnp.float32
_PH = lax.Precision.HIGHEST


def _sc_body(ei_hbm, ew_hbm, out_hbm, src_v, dst_v, w_v, idx_v, z_v,
             acc_sh, sem):
    s = lax.axis_index("s")
    for i in range(_SLICE // 16):
        z_v[pl.ds(i * 16, 16)] = jnp.zeros((16,), _F32)
    pltpu.sync_copy(z_v, acc_sh.at[pl.ds(s * _SLICE, _SLICE)])
    plsc.subcore_barrier()

    def fetch_pair(half, r0):
        o = half * 2 * _EC
        sl = pl.ds(r0 * _EC, 2 * _EC)
        return [
            pltpu.async_copy(ei_hbm.at[0, 0, sl], src_v.at[pl.ds(o, 2 * _EC)], sem),
            pltpu.async_copy(ei_hbm.at[0, 1, sl], dst_v.at[pl.ds(o, 2 * _EC)], sem),
            pltpu.async_copy(ew_hbm.at[0, sl], w_v.at[pl.ds(o, 2 * _EC)], sem),
        ]

    def scatter_rows(j0):
        for j in (j0, j0 + 1):
            for k in range(_EC // 16):
                sl = pl.ds(j * _EC + k * 16, 16)
                idx_v[j, pl.ds(k * 16, 16)] = dst_v[sl] * _NP + src_v[sl]
            pltpu.sync_copy(w_v.at[pl.ds(j * _EC, _EC)],
                            acc_sh.at[idx_v.at[j]], add=True)

    second = s < _ER // 2 - 16
    cps0 = fetch_pair(0, 2 * s)

    @pl.when(second)
    def _():
        fetch_pair(1, 32 + 2 * s)

    for cp in cps0:
        cp.wait()
    scatter_rows(0)

    @pl.when(second)
    def _():
        o = 2 * _EC
        sl = pl.ds((32 + 2 * s) * _EC, 2 * _EC)
        pltpu.make_async_copy(ei_hbm.at[0, 0, sl],
                              src_v.at[pl.ds(o, 2 * _EC)], sem).wait()
        pltpu.make_async_copy(ei_hbm.at[0, 1, sl],
                              dst_v.at[pl.ds(o, 2 * _EC)], sem).wait()
        pltpu.make_async_copy(ew_hbm.at[0, sl],
                              w_v.at[pl.ds(o, 2 * _EC)], sem).wait()
        scatter_rows(2)

    plsc.subcore_barrier()
    ocps = [
        pltpu.async_copy(acc_sh.at[pl.ds(s * _SLICE + k * _NP, _NP)],
                         out_hbm.at[s, k, pl.ds(0, _NP)], sem)
        for k in range(8)
    ]
    for cp in ocps:
        cp.wait()


_sc_build = pl.kernel(
    _sc_body,
    mesh=plsc.VectorSubcoreMesh(core_axis_name="c", subcore_axis_name="s",
                                num_cores=1),
    out_type=jax.ShapeDtypeStruct((16, 8, _NP), _F32),
    scratch_types=[
        pltpu.VMEM((4 * _EC,), jnp.int32),
        pltpu.VMEM((4 * _EC,), jnp.int32),
        pltpu.VMEM((4 * _EC,), _F32),
        pltpu.VMEM((4, _EC), jnp.int32),
        pltpu.VMEM((_SLICE,), _F32),
        pltpu.VMEM_SHARED((_ACC,), _F32),
        pltpu.SemaphoreType.DMA,
    ],
)


def _dense_body(a_ref, x_ref, wg1_ref, bg1_ref,
                wg2_ref, bg2_ref, wg3_ref, bg3_ref, gam_ref, bet_ref,
                w1_ref, b1_ref, w2_ref, b2_ref, out_ref):
    at = a_ref[...].reshape(_NP, _NP)
    deg = jnp.sum(at, axis=1, keepdims=True) + 1.0
    dinv = lax.rsqrt(deg)
    xp = jnp.concatenate(
        [x_ref[0], jnp.zeros((_NP - _N, 128), _F32)], axis=0)

    def gcn(h, w, b):
        z = jnp.dot(h, w, preferred_element_type=_F32)
        zh = dinv * z
        agg = jnp.dot(at, zh, precision=_PH, preferred_element_type=_F32) + zh
        return jax.nn.relu(dinv * agg + b)

    h = gcn(xp, wg1_ref[...], bg1_ref[...])
    h = gcn(h, wg2_ref[...], bg2_ref[...])
    h = gcn(h, wg3_ref[...], bg3_ref[...])

    rmask = (lax.broadcasted_iota(jnp.int32, (_NP, 1), 0) < _N).astype(_F32)
    mean = jnp.sum(h * rmask, axis=0, keepdims=True) * (1.0 / _N)
    diff = h - mean
    var = jnp.sum(diff * diff * rmask, axis=0, keepdims=True) * (1.0 / _N)
    hn = diff * lax.rsqrt(var + 1e-5) * gam_ref[...] + bet_ref[...]

    l = jax.nn.relu(jnp.dot(hn, w1_ref[...],
                            preferred_element_type=_F32) + b1_ref[...])
    r_i = lax.broadcasted_iota(jnp.int32, (1000, _N), 0)
    i_i = lax.broadcasted_iota(jnp.int32, (1000, _N), 1)
    t = r_i - 10 * i_i
    nsel = ((t >= 0) & (t < 10)).astype(_F32)
    r_c = lax.broadcasted_iota(jnp.int32, (1000, 10), 0)
    c_c = lax.broadcasted_iota(jnp.int32, (1000, 10), 1)
    qsel = (r_c % 10 == c_c).astype(_F32)
    lrows = jnp.dot(nsel, l[:_N, :], preferred_element_type=_F32)
    sflat = jnp.sum(lrows * qsel, axis=1, keepdims=True)
    out = jnp.sum(sflat * w2_ref[...], axis=0, keepdims=True)
    out_ref[...] = out[0] + b2_ref[...]


def kernel(x, edge_index, edge_attr, Wg1, bg1, Wg2, bg2, Wg3, bg3,
           gamma, beta, W1, b1, W2, b2):
    a = _sc_build(edge_index.astype(jnp.int32), edge_attr)
    return pl.pallas_call(
        _dense_body,
        out_shape=jax.ShapeDtypeStruct((128,), _F32),
    )(a, x, Wg1, bg1, Wg2, bg2, Wg3, bg3, gamma, beta, W1, b1, W2, b2)


# speedup vs baseline: 1.1990x; 1.0073x over previous
import jax
import jax.numpy as jnp
from jax import lax
from jax.experimental import pallas as pl
from jax.experimental.pallas import tpu as pltpu
from jax.experimental.pallas import tpu_sc as plsc

_N = 100
_NP = 128
_E = 6400
_ER = 50
_EC = 128
_ACC = _NP * _NP
_SLICE = _ACC // 16
_F32 = jnp.float32
_PH = lax.Precision.HIGHEST


def _sc_body(ei_hbm, ew_hbm, out_hbm, src_v, dst_v, w_v, idx_v, z_v,
             acc_sh, sem):
    s = lax.axis_index("s")
    for i in range(_SLICE // 16):
        z_v[pl.ds(i * 16, 16)] = jnp.zeros((16,), _F32)
    pltpu.sync_copy(z_v, acc_sh.at[pl.ds(s * _SLICE, _SLICE)])
    plsc.subcore_barrier()

    def fetch_pair(half, r0):
        o = half * 2 * _EC
        sl = pl.ds(r0 * _EC, 2 * _EC)
        dsl = pl.ds(_E + r0 * _EC, 2 * _EC)
        return [
            pltpu.async_copy(ei_hbm.at[sl], src_v.at[pl.ds(o, 2 * _EC)], sem),
            pltpu.async_copy(ei_hbm.at[dsl], dst_v.at[pl.ds(o, 2 * _EC)], sem),
            pltpu.async_copy(ew_hbm.at[sl], w_v.at[pl.ds(o, 2 * _EC)], sem),
        ]

    def scatter_rows(j0):
        for j in (j0, j0 + 1):
            for k in range(_EC // 16):
                sl = pl.ds(j * _EC + k * 16, 16)
                idx_v[j, pl.ds(k * 16, 16)] = dst_v[sl] * _NP + src_v[sl]
            pltpu.sync_copy(w_v.at[pl.ds(j * _EC, _EC)],
                            acc_sh.at[idx_v.at[j]], add=True)

    second = s < _ER // 2 - 16
    cps0 = fetch_pair(0, 2 * s)

    @pl.when(second)
    def _():
        fetch_pair(1, 32 + 2 * s)

    for cp in cps0:
        cp.wait()
    scatter_rows(0)

    @pl.when(second)
    def _():
        o = 2 * _EC
        sl = pl.ds((32 + 2 * s) * _EC, 2 * _EC)
        dsl = pl.ds(_E + (32 + 2 * s) * _EC, 2 * _EC)
        pltpu.make_async_copy(ei_hbm.at[sl],
                              src_v.at[pl.ds(o, 2 * _EC)], sem).wait()
        pltpu.make_async_copy(ei_hbm.at[dsl],
                              dst_v.at[pl.ds(o, 2 * _EC)], sem).wait()
        pltpu.make_async_copy(ew_hbm.at[sl],
                              w_v.at[pl.ds(o, 2 * _EC)], sem).wait()
        scatter_rows(2)

    plsc.subcore_barrier()
    ocps = [
        pltpu.async_copy(acc_sh.at[pl.ds(s * _SLICE + k * _NP, _NP)],
                         out_hbm.at[s, k, pl.ds(0, _NP)], sem)
        for k in range(8)
    ]
    for cp in ocps:
        cp.wait()


_sc_build = pl.kernel(
    _sc_body,
    mesh=plsc.VectorSubcoreMesh(core_axis_name="c", subcore_axis_name="s",
                                num_cores=1),
    out_type=jax.ShapeDtypeStruct((16, 8, _NP), _F32),
    scratch_types=[
        pltpu.VMEM((4 * _EC,), jnp.int32),
        pltpu.VMEM((4 * _EC,), jnp.int32),
        pltpu.VMEM((4 * _EC,), _F32),
        pltpu.VMEM((4, _EC), jnp.int32),
        pltpu.VMEM((_SLICE,), _F32),
        pltpu.VMEM_SHARED((_ACC,), _F32),
        pltpu.SemaphoreType.DMA,
    ],
)


def _dense_body(a_ref, x_ref, wg1_ref, bg1_ref,
                wg2_ref, bg2_ref, wg3_ref, bg3_ref, gam_ref, bet_ref,
                w1_ref, b1_ref, w2_ref, b2_ref, out_ref):
    at = a_ref[...].reshape(_NP, _NP)
    deg = jnp.sum(at, axis=1, keepdims=True) + 1.0
    dinv = lax.rsqrt(deg)
    xp = jnp.concatenate(
        [x_ref[0], jnp.zeros((_NP - _N, 128), _F32)], axis=0)

    def gcn(h, w, b):
        z = jnp.dot(h, w, preferred_element_type=_F32)
        zh = dinv * z
        agg = jnp.dot(at, zh, precision=_PH, preferred_element_type=_F32) + zh
        return jax.nn.relu(dinv * agg + b)

    h = gcn(xp, wg1_ref[...], bg1_ref[...])
    h = gcn(h, wg2_ref[...], bg2_ref[...])
    h = gcn(h, wg3_ref[...], bg3_ref[...])

    rmask = (lax.broadcasted_iota(jnp.int32, (_NP, 1), 0) < _N).astype(_F32)
    mean = jnp.sum(h * rmask, axis=0, keepdims=True) * (1.0 / _N)
    diff = h - mean
    var = jnp.sum(diff * diff * rmask, axis=0, keepdims=True) * (1.0 / _N)
    hn = diff * lax.rsqrt(var + 1e-5) * gam_ref[...] + bet_ref[...]

    l = jax.nn.relu(jnp.dot(hn, w1_ref[...],
                            preferred_element_type=_F32) + b1_ref[...])
    r_i = lax.broadcasted_iota(jnp.int32, (1000, _N), 0)
    i_i = lax.broadcasted_iota(jnp.int32, (1000, _N), 1)
    t = r_i - 10 * i_i
    nsel = ((t >= 0) & (t < 10)).astype(_F32)
    r_c = lax.broadcasted_iota(jnp.int32, (1000, 10), 0)
    c_c = lax.broadcasted_iota(jnp.int32, (1000, 10), 1)
    qsel = (r_c % 10 == c_c).astype(_F32)
    lrows = jnp.dot(nsel, l[:_N, :], preferred_element_type=_F32)
    sflat = jnp.sum(lrows * qsel, axis=1, keepdims=True)
    out = jnp.sum(sflat * w2_ref[...], axis=0, keepdims=True)
    out_ref[...] = out[0] + b2_ref[...]


def kernel(x, edge_index, edge_attr, Wg1, bg1, Wg2, bg2, Wg3, bg3,
           gamma, beta, W1, b1, W2, b2):
    a = _sc_build(edge_index.astype(jnp.int32).reshape(2 * _E),
                  edge_attr.reshape(_E))
    return pl.pallas_call(
        _dense_body,
        out_shape=jax.ShapeDtypeStruct((128,), _F32),
    )(a, x, Wg1, bg1, Wg2, bg2, Wg3, bg3, gamma, beta, W1, b1, W2, b2)
